# Initial kernel scaffold; baseline (speedup 1.0000x reference)
import jax
import jax.numpy as jnp
from jax.experimental import pallas as pl


def kernel(stamp, table):
    raise NotImplementedError("write your pallas kernel here")


# speedup vs baseline: 4.0216x; 4.0216x over previous
import functools

import jax
import jax.numpy as jnp
import numpy as np
from jax import lax
from jax.experimental import pallas as pl
from jax.experimental.pallas import tpu as pltpu
from jax.experimental.pallas import tpu_sc as plsc

CIRCLE = 1000
D = 64
N_HIS = 20
BATCH = 16384
B_FLAT = BATCH * N_HIS

NC = 2
NS = 16
NW = NC * NS
PW = B_FLAT // NW
C = 128
NCHUNK = PW // C


def _tempo_enc_np(n_his, d):
    pos = np.arange(n_his, dtype=np.float64)[:, None]
    i = np.arange(d, dtype=np.float64)[None, :]
    angle = pos / np.power(10000.0, (2.0 * (i // 2)) / d)
    enc = np.zeros((n_his, d), dtype=np.float64)
    enc[:, 0::2] = np.sin(angle[:, 0::2])
    enc[:, 1::2] = np.cos(angle[:, 1::2])
    return enc.astype(np.float32)


_ENC = _tempo_enc_np(N_HIS, D)


def _prep_body(stamp_ref, table_ref, enc_ref, exp_ref, idx_ref):
    t = table_ref[...]
    enc = enc_ref[...]
    exp_ref[...] = t[:, None, :] + enc[None, :, :]
    h = lax.broadcasted_iota(jnp.int32, (BATCH, N_HIS), 1)
    idx_ref[...] = stamp_ref[...] * N_HIS + h


def _prep(stamp, table):
    return pl.pallas_call(
        _prep_body,
        out_shape=(
            jax.ShapeDtypeStruct((CIRCLE, N_HIS, D), jnp.float32),
            jax.ShapeDtypeStruct((BATCH, N_HIS), jnp.int32),
        ),
    )(stamp, table, jnp.asarray(_ENC))


def _gather_body(exp_hbm, idx_hbm, out_hbm, idx_v, buf0, buf1, sem0, sem1):
    w = lax.axis_index("s") * NC + lax.axis_index("c")
    base = w * PW
    pltpu.sync_copy(idx_hbm.at[w], idx_v)
    pltpu.async_copy(exp_hbm.at[idx_v.at[0]], buf0, sem0)

    def pair(p, carry):
        c0 = 2 * p
        pltpu.make_async_copy(exp_hbm.at[idx_v.at[c0]], buf0, sem0).wait()
        pltpu.async_copy(exp_hbm.at[idx_v.at[c0 + 1]], buf1, sem1)
        pltpu.sync_copy(buf0, out_hbm.at[pl.ds(base + c0 * C, C)])
        pltpu.make_async_copy(exp_hbm.at[idx_v.at[c0 + 1]], buf1, sem1).wait()

        @pl.when(p < NCHUNK // 2 - 1)
        def _():
            pltpu.async_copy(exp_hbm.at[idx_v.at[c0 + 2]], buf0, sem0)

        pltpu.sync_copy(buf1, out_hbm.at[pl.ds(base + (c0 + 1) * C, C)])
        return carry

    lax.fori_loop(0, NCHUNK // 2, pair, 0)


_gather = functools.partial(
    pl.kernel,
    out_type=jax.ShapeDtypeStruct((B_FLAT, D), jnp.float32),
    mesh=plsc.VectorSubcoreMesh(core_axis_name="c", subcore_axis_name="s"),
    scratch_types=[
        pltpu.VMEM((NCHUNK, C), jnp.int32),
        pltpu.VMEM((C, D), jnp.float32),
        pltpu.VMEM((C, D), jnp.float32),
        pltpu.SemaphoreType.DMA,
        pltpu.SemaphoreType.DMA,
    ],
    compiler_params=pltpu.CompilerParams(use_tc_tiling_on_sc=False),
)(_gather_body)


def kernel(stamp, table):
    exp, idx = _prep(stamp, table)
    exp_flat = exp.reshape(CIRCLE * N_HIS, D)
    idx_w = idx.reshape(NW, NCHUNK, C)
    out = _gather(exp_flat, idx_w)
    return out.reshape(BATCH, N_HIS, D)


# speedup vs baseline: 4.7776x; 1.1880x over previous
import functools

import jax
import jax.numpy as jnp
import numpy as np
from jax import lax
from jax.experimental import pallas as pl
from jax.experimental.pallas import tpu as pltpu
from jax.experimental.pallas import tpu_sc as plsc

CIRCLE = 1000
D = 64
N_HIS = 20
BATCH = 16384
B_FLAT = BATCH * N_HIS

NC = 2
NS = 16
NW = NC * NS
PW = B_FLAT // NW
C = 128
NCHUNK = PW // C


def _tempo_enc_np(n_his, d):
    pos = np.arange(n_his, dtype=np.float64)[:, None]
    i = np.arange(d, dtype=np.float64)[None, :]
    angle = pos / np.power(10000.0, (2.0 * (i // 2)) / d)
    enc = np.zeros((n_his, d), dtype=np.float64)
    enc[:, 0::2] = np.sin(angle[:, 0::2])
    enc[:, 1::2] = np.cos(angle[:, 1::2])
    return enc.astype(np.float32)


_ENC = _tempo_enc_np(N_HIS, D)


def _prep_body(stamp_ref, table_ref, enc_ref, exp_ref, idx_ref):
    t = table_ref[...]
    enc = enc_ref[...]
    exp_ref[...] = t[:, None, :] + enc[None, :, :]
    h = lax.broadcasted_iota(jnp.int32, (BATCH, N_HIS), 1)
    idx_ref[...] = stamp_ref[...] * N_HIS + h


def _prep(stamp, table):
    return pl.pallas_call(
        _prep_body,
        out_shape=(
            jax.ShapeDtypeStruct((CIRCLE, N_HIS, D), jnp.float32),
            jax.ShapeDtypeStruct((BATCH, N_HIS), jnp.int32),
        ),
    )(stamp, table, jnp.asarray(_ENC))


def _gather_body(exp_hbm, idx_hbm, out_hbm, idx_v, buf0, buf1, sem0, sem1):
    w = lax.axis_index("s") * NC + lax.axis_index("c")
    base = w * PW
    pltpu.sync_copy(idx_hbm.at[w], idx_v)
    pltpu.async_copy(exp_hbm.at[idx_v.at[0]], buf0, sem0)

    def pair(p, carry):
        c0 = 2 * p
        pltpu.make_async_copy(exp_hbm.at[idx_v.at[c0]], buf0, sem0).wait()
        pltpu.async_copy(exp_hbm.at[idx_v.at[c0 + 1]], buf1, sem1)
        pltpu.sync_copy(buf0, out_hbm.at[pl.ds(base + c0 * C, C)])
        pltpu.make_async_copy(exp_hbm.at[idx_v.at[c0 + 1]], buf1, sem1).wait()

        @pl.when(p < NCHUNK // 2 - 1)
        def _():
            pltpu.async_copy(exp_hbm.at[idx_v.at[c0 + 2]], buf0, sem0)

        pltpu.sync_copy(buf1, out_hbm.at[pl.ds(base + (c0 + 1) * C, C)])
        return carry

    lax.fori_loop(0, NCHUNK // 2, pair, 0)


_gather = functools.partial(
    pl.kernel,
    out_type=jax.ShapeDtypeStruct((B_FLAT, D), jnp.float32),
    mesh=plsc.VectorSubcoreMesh(core_axis_name="c", subcore_axis_name="s"),
    scratch_types=[
        pltpu.VMEM((NCHUNK, C), jnp.int32),
        pltpu.VMEM((C, D), jnp.float32),
        pltpu.VMEM((C, D), jnp.float32),
        pltpu.SemaphoreType.DMA,
        pltpu.SemaphoreType.DMA,
    ],
    compiler_params=pltpu.CompilerParams(use_tc_tiling_on_sc=False),
)(_gather_body)


BB = 1024


def _tr_body(x_ref, o_ref):
    x = x_ref[...]
    for h in range(N_HIS):
        o_ref[h, :, :] = x[:, h * D:(h + 1) * D].T


def _transpose(x2):
    return pl.pallas_call(
        _tr_body,
        grid=(BATCH // BB,),
        in_specs=[pl.BlockSpec((BB, N_HIS * D), lambda i: (i, 0))],
        out_specs=pl.BlockSpec((N_HIS, D, BB), lambda i: (0, 0, i)),
        out_shape=jax.ShapeDtypeStruct((N_HIS, D, BATCH), jnp.float32),
    )(x2)


def kernel(stamp, table):
    exp, idx = _prep(stamp, table)
    exp_flat = exp.reshape(CIRCLE * N_HIS, D)
    idx_w = idx.reshape(NW, NCHUNK, C)
    out = _gather(exp_flat, idx_w)
    y = _transpose(out.reshape(BATCH, N_HIS * D))
    return jnp.transpose(y, (2, 0, 1))
